# Initial kernel scaffold; baseline (speedup 1.0000x reference)
import jax
import jax.numpy as jnp
from jax.experimental import pallas as pl


def kernel(x, edge_index, W1, b1, W2, b2):
    raise NotImplementedError("write your pallas kernel here")


# speedup vs baseline: 6.5294x; 6.5294x over previous
import functools

import jax
import jax.numpy as jnp
from jax import lax
from jax.experimental import pallas as pl
from jax.experimental.pallas import tpu as pltpu
from jax.experimental.pallas import tpu_sc as plsc

N = 10000
E = 320000
D = 128
K = 10
ALPHA = 0.1

CH = 128
TILES = 32
CHUNKS_PER_TILE = 79
EP_TILE = CHUNKS_PER_TILE * CH
E_PAD = EP_TILE * TILES
NP = 10240
SAC = 10008
ROWS_T = NP // 16

_mesh = plsc.VectorSubcoreMesh(core_axis_name="c", subcore_axis_name="s")


@functools.partial(
    pl.kernel,
    mesh=_mesh,
    out_type=[
        jax.ShapeDtypeStruct((NP, 16), jnp.float32),
        jax.ShapeDtypeStruct((NP, 16), jnp.float32),
    ],
    scratch_types=[
        pltpu.VMEM_SHARED((NP, 16), jnp.float32),
        pltpu.VMEM((CH,), jnp.int32),
        pltpu.VMEM((CH, 16), jnp.float32),
    ],
)
def _deg_kernel(dst_hbm, ones_hbm, degA, degB, sh_deg, idx_v, ones_v):
    c = lax.axis_index("c")
    s = lax.axis_index("s")
    wid = c * 16 + s
    pltpu.sync_copy(ones_hbm.at[pl.ds(s * ROWS_T, ROWS_T)],
                    sh_deg.at[pl.ds(s * ROWS_T, ROWS_T)])
    pltpu.sync_copy(ones_hbm.at[pl.ds(0, CH)], ones_v)
    plsc.subcore_barrier()
    base0 = wid * EP_TILE

    def body(j, carry):
        pltpu.sync_copy(dst_hbm.at[pl.ds(base0 + j * CH, CH)], idx_v)
        pltpu.sync_copy(ones_v, sh_deg.at[idx_v], add=True)
        return carry

    lax.fori_loop(0, CHUNKS_PER_TILE, body, 0)
    plsc.subcore_barrier()

    @pl.when(c == 0)
    def _():
        pltpu.sync_copy(sh_deg.at[pl.ds(s * ROWS_T, ROWS_T)],
                        degA.at[pl.ds(s * ROWS_T, ROWS_T)])

    @pl.when(c == 1)
    def _():
        pltpu.sync_copy(sh_deg.at[pl.ds(s * ROWS_T, ROWS_T)],
                        degB.at[pl.ds(s * ROWS_T, ROWS_T)])


@functools.partial(
    pl.kernel,
    mesh=_mesh,
    out_type=[
        jax.ShapeDtypeStruct((NP, D), jnp.float32),
        jax.ShapeDtypeStruct((NP, D), jnp.float32),
    ],
    scratch_types=[
        pltpu.VMEM_SHARED((NP, D), jnp.float32),
        pltpu.VMEM((CH,), jnp.int32),
        pltpu.VMEM((CH,), jnp.int32),
        pltpu.VMEM((CH, D), jnp.float32),
        pltpu.SemaphoreType.DMA,
    ],
)
def _scatter_step(g_hbm, src_hbm, dst_hbm, aggA, aggB,
                  sh_agg, idx_s, idx_d, rows, sem):
    c = lax.axis_index("c")
    s = lax.axis_index("s")
    wid = c * 16 + s
    pltpu.sync_copy(g_hbm.at[pl.ds(s * ROWS_T, ROWS_T)],
                    sh_agg.at[pl.ds(s * ROWS_T, ROWS_T)])
    plsc.subcore_barrier()
    base0 = wid * EP_TILE

    def body(j, carry):
        b = base0 + j * CH
        pltpu.sync_copy(src_hbm.at[pl.ds(b, CH)], idx_s)
        pltpu.sync_copy(dst_hbm.at[pl.ds(b, CH)], idx_d)
        pltpu.async_copy(g_hbm.at[idx_s], rows, sem).wait()
        pltpu.sync_copy(rows, sh_agg.at[idx_d], add=True)
        return carry

    lax.fori_loop(0, CHUNKS_PER_TILE, body, 0)
    plsc.subcore_barrier()

    @pl.when(c == 0)
    def _():
        pltpu.sync_copy(sh_agg.at[pl.ds(s * ROWS_T, ROWS_T)],
                        aggA.at[pl.ds(s * ROWS_T, ROWS_T)])

    @pl.when(c == 1)
    def _():
        pltpu.sync_copy(sh_agg.at[pl.ds(s * ROWS_T, ROWS_T)],
                        aggB.at[pl.ds(s * ROWS_T, ROWS_T)])


def _mlp_body(x_ref, w1_ref, b1_ref, w2_ref, b2_ref, da_ref, db_ref,
              x0_ref, g0_ref, dinv_ref):
    xb = x_ref[...]
    h = lax.dot_general(xb, w1_ref[...], (((1,), (1,)), ((), ())),
                        preferred_element_type=jnp.float32)
    h = jnp.maximum(h + b1_ref[...], 0.0)
    h = lax.dot_general(h, w2_ref[...], (((1,), (1,)), ((), ())),
                        preferred_element_type=jnp.float32) + b2_ref[...]
    deg = da_ref[...][:, :1] + db_ref[...][:, :1] - 1.0
    dinv = lax.rsqrt(deg)
    x0_ref[...] = h
    g0_ref[...] = h * dinv
    dinv_ref[...] = dinv


def _combine_body(aggA_ref, aggB_ref, g_ref, x0_ref, dinv_ref, h_ref, gn_ref):
    agg = aggA_ref[...] + aggB_ref[...] - g_ref[...]
    dinv = dinv_ref[...]
    h = (1.0 - ALPHA) * (dinv * agg) + ALPHA * x0_ref[...]
    h_ref[...] = h
    gn_ref[...] = dinv * h


_BLK = 1024
_GRID = NP // _BLK

_mlp_call = pl.pallas_call(
    _mlp_body,
    grid=(_GRID,),
    in_specs=[
        pl.BlockSpec((_BLK, D), lambda i: (i, 0)),
        pl.BlockSpec((D, D), lambda i: (0, 0)),
        pl.BlockSpec((1, D), lambda i: (0, 0)),
        pl.BlockSpec((D, D), lambda i: (0, 0)),
        pl.BlockSpec((1, D), lambda i: (0, 0)),
        pl.BlockSpec((_BLK, 16), lambda i: (i, 0)),
        pl.BlockSpec((_BLK, 16), lambda i: (i, 0)),
    ],
    out_specs=[
        pl.BlockSpec((_BLK, D), lambda i: (i, 0)),
        pl.BlockSpec((_BLK, D), lambda i: (i, 0)),
        pl.BlockSpec((_BLK, 1), lambda i: (i, 0)),
    ],
    out_shape=[
        jax.ShapeDtypeStruct((NP, D), jnp.float32),
        jax.ShapeDtypeStruct((NP, D), jnp.float32),
        jax.ShapeDtypeStruct((NP, 1), jnp.float32),
    ],
)

_combine_call = pl.pallas_call(
    _combine_body,
    grid=(_GRID,),
    in_specs=[
        pl.BlockSpec((_BLK, D), lambda i: (i, 0)),
        pl.BlockSpec((_BLK, D), lambda i: (i, 0)),
        pl.BlockSpec((_BLK, D), lambda i: (i, 0)),
        pl.BlockSpec((_BLK, D), lambda i: (i, 0)),
        pl.BlockSpec((_BLK, 1), lambda i: (i, 0)),
    ],
    out_specs=[
        pl.BlockSpec((_BLK, D), lambda i: (i, 0)),
        pl.BlockSpec((_BLK, D), lambda i: (i, 0)),
    ],
    out_shape=[
        jax.ShapeDtypeStruct((NP, D), jnp.float32),
        jax.ShapeDtypeStruct((NP, D), jnp.float32),
    ],
)


def kernel(x, edge_index, W1, b1, W2, b2):
    src = edge_index[0]
    dst = edge_index[1]
    pad = E_PAD - E
    src_p = jnp.concatenate([src, jnp.zeros((pad,), jnp.int32)])
    dst_p = jnp.concatenate([dst, jnp.full((pad,), SAC, jnp.int32)])
    ones16 = jnp.ones((NP, 16), jnp.float32)
    x_p = jnp.pad(x, ((0, NP - N), (0, 0)))

    degA, degB = _deg_kernel(dst_p, ones16)
    x0, g, dinv = _mlp_call(x_p, W1, b1.reshape(1, D), W2, b2.reshape(1, D),
                            degA, degB)
    h = x0
    for _ in range(K):
        aggA, aggB = _scatter_step(g, src_p, dst_p)
        h, g = _combine_call(aggA, aggB, g, x0, dinv)
    return h[:N]
